# Initial kernel scaffold; baseline (speedup 1.0000x reference)
import jax
import jax.numpy as jnp
from jax.experimental import pallas as pl


def kernel(features, edge_index, W1, b1, Wl1, bl1, gamma, beta, W2, b2, Wl2, bl2):
    raise NotImplementedError("write your pallas kernel here")


# speedup vs baseline: 13.0676x; 13.0676x over previous
import jax
import jax.numpy as jnp
from jax import lax
from jax.experimental import pallas as pl
from jax.experimental.pallas import tpu as pltpu
from jax.experimental.pallas import tpu_sc as plsc

N = 10000
E = 320000
D_IN = 128
D_HID = 256
D_OUT = 128

NC = 2
NS = 16
NW = NC * NS
EPT = E // NW
C = 80
NCHUNK = EPT // C
NP = 10240
RPT = NP // NS
GC = 25
NG = NCHUNK // GC

_F32 = jnp.float32


def _make_seg_sum(with_deg):
    out_type = [jax.ShapeDtypeStruct((NC, NP, D_IN), _F32)]
    if with_deg:
        out_type.append(jax.ShapeDtypeStruct((NC * NP,), _F32))
    scratch = [
        pltpu.VMEM_SHARED((NP, D_IN), _F32),
        pltpu.VMEM((RPT // C, C), jnp.int32),
        pltpu.VMEM((GC, C), jnp.int32),
        pltpu.VMEM((GC, C), jnp.int32),
        pltpu.VMEM((C, D_IN), _F32),
        pltpu.VMEM((C, D_IN), _F32),
        pltpu.VMEM((C, D_IN), _F32),
        pltpu.SemaphoreType.DMA,
        pltpu.SemaphoreType.DMA,
        pltpu.SemaphoreType.DMA,
        pltpu.SemaphoreType.DMA,
        pltpu.SemaphoreType.DMA,
        pltpu.SemaphoreType.DMA,
        pltpu.SemaphoreType.DMA,
    ]
    if with_deg:
        scratch.append(pltpu.VMEM_SHARED((NP,), _F32))
        scratch.append(pltpu.VMEM((C,), _F32))

    mesh = plsc.VectorSubcoreMesh(core_axis_name="c", subcore_axis_name="s")

    def body(table, rows, srcs, dsts, *rest):
        if with_deg:
            (agg_out, deg_out, acc, rows_v, src_v, dst_v, b0, b1, b2,
             g0, g1, g2, s0, s1, s2, sd, deg_acc, ones_v) = rest
        else:
            (agg_out, acc, rows_v, src_v, dst_v, b0, b1, b2,
             g0, g1, g2, s0, s1, s2, sd) = rest
        bufs = (b0, b1, b2)
        gsem = (g0, g1, g2)
        ssem = (s0, s1, s2)
        buf = b0
        c = lax.axis_index("c")
        s = lax.axis_index("s")
        w = c * NS + s
        r0 = s * RPT
        npiece = RPT // C

        zvec = jnp.zeros((16,), _F32)
        ones = jnp.ones((16,), _F32)

        pltpu.sync_copy(rows.at[s], rows_v)

        def zrow(i, carry):
            for j in range(D_IN // 16):
                buf[i, pl.ds(j * 16, 16)] = zvec
            return carry

        lax.fori_loop(0, C, zrow, 0)
        for k in range(npiece):
            pltpu.sync_copy(buf, acc.at[rows_v.at[k]])

        if with_deg:
            for j in range(C // 16):
                ones_v[pl.ds(j * 16, 16)] = zvec
            for k in range(npiece):
                pltpu.sync_copy(ones_v, deg_acc.at[rows_v.at[k]])
            for j in range(C // 16):
                ones_v[pl.ds(j * 16, 16)] = ones

        plsc.subcore_barrier()

        def gstart(k, j):
            pltpu.async_copy(table.at[src_v.at[k]], bufs[j], gsem[j])

        def gwait(j):
            pltpu.make_async_copy(table.at[src_v.at[0]], bufs[j],
                                  gsem[j]).wait()

        def sstart(k, j):
            pltpu.async_copy(bufs[j], acc.at[dst_v.at[k]], ssem[j], add=True)
            if with_deg:
                pltpu.async_copy(ones_v, deg_acc.at[dst_v.at[k]], sd,
                                 add=True)

        def swait(j):
            pltpu.make_async_copy(bufs[j], acc.at[dst_v.at[0]],
                                  ssem[j]).wait()

        def slot(k, j, start_next, wait_prev):
            gwait(j)
            sstart(k, j)
            if wait_prev:
                swait((j + 2) % 3)
            if start_next:
                gstart(k + 2, (j + 2) % 3)

        def group(g, carry):
            pltpu.sync_copy(srcs.at[w, g], src_v)
            pltpu.sync_copy(dsts.at[w, g], dst_v)
            gstart(0, 0)
            gstart(1, 1)
            gwait(0); sstart(0, 0); gstart(2, 2)
            gwait(1); sstart(1, 1); swait(0); gstart(3, 0)
            gwait(2); sstart(2, 2); swait(1); gstart(4, 1)

            def triple(p, carry2):
                k0 = 3 * p
                for j in range(3):
                    slot(k0 + j, j, True, True)
                return carry2

            lax.fori_loop(1, (GC - 4) // 3, triple, 0)
            slot(GC - 4, 0, True, True)
            slot(GC - 3, 1, True, True)
            slot(GC - 2, 2, False, True)
            slot(GC - 1, 0, False, True)
            swait(0)
            if with_deg:
                for _ in range(GC):
                    pltpu.make_async_copy(ones_v, deg_acc.at[dst_v.at[0]],
                                          sd).wait()
            return carry

        lax.fori_loop(0, NG, group, 0)

        plsc.subcore_barrier()

        for k in range(npiece):
            pltpu.sync_copy(acc.at[rows_v.at[k]], buf)
            pltpu.sync_copy(buf, agg_out.at[c, pl.ds(r0 + k * C, C), :])
        if with_deg:
            for k in range(npiece):
                pltpu.sync_copy(deg_acc.at[rows_v.at[k]], ones_v)
                pltpu.sync_copy(ones_v,
                                deg_out.at[pl.ds(c * NP + r0 + k * C, C)])

    return pl.kernel(body, out_type=out_type, mesh=mesh,
                     scratch_types=scratch)


_seg_sum_deg = _make_seg_sum(True)
_seg_sum = _make_seg_sum(False)

_R = 1000
_G = N // _R


def _dense_a_body(x, a0, a1, d0, d1, w1, wl1, b, h1_ref, s_ref, q_ref):
    scale = 1.0 / (d0[0] + d1[0] + 1.0)
    hn = (a0[0] + a1[0] + x[...]) * scale
    h1 = (jnp.dot(hn, w1[...], preferred_element_type=_F32)
          + jnp.dot(x[...], wl1[...], preferred_element_type=_F32)
          + b[...])
    h1_ref[...] = h1

    @pl.when(pl.program_id(0) == 0)
    def _():
        s_ref[...] = jnp.zeros_like(s_ref)
        q_ref[...] = jnp.zeros_like(q_ref)

    s_ref[...] += jnp.sum(h1, axis=0, keepdims=True)
    q_ref[...] += jnp.sum(h1 * h1, axis=0, keepdims=True)


def _dense_a(x, a0, a1, d0, d1, W1, Wl1, bias1):
    return pl.pallas_call(
        _dense_a_body,
        grid=(_G,),
        in_specs=[
            pl.BlockSpec((_R, D_IN), lambda i: (i, 0)),
            pl.BlockSpec((1, _R, D_IN), lambda i: (0, i, 0)),
            pl.BlockSpec((1, _R, D_IN), lambda i: (1, i, 0)),
            pl.BlockSpec((1, _R, 1), lambda i: (0, i, 0)),
            pl.BlockSpec((1, _R, 1), lambda i: (1, i, 0)),
            pl.BlockSpec((D_IN, D_HID), lambda i: (0, 0)),
            pl.BlockSpec((D_IN, D_HID), lambda i: (0, 0)),
            pl.BlockSpec((1, D_HID), lambda i: (0, 0)),
        ],
        out_specs=[
            pl.BlockSpec((_R, D_HID), lambda i: (i, 0)),
            pl.BlockSpec((1, D_HID), lambda i: (0, 0)),
            pl.BlockSpec((1, D_HID), lambda i: (0, 0)),
        ],
        out_shape=[
            jax.ShapeDtypeStruct((N, D_HID), _F32),
            jax.ShapeDtypeStruct((1, D_HID), _F32),
            jax.ShapeDtypeStruct((1, D_HID), _F32),
        ],
    )(x, a0, a1, d0, d1, W1, Wl1, bias1)


def _dense_b_body(h1, s_ref, q_ref, g, bt, w2, wl2, b2l, y_ref, z_ref):
    mean = s_ref[...] / N
    var = q_ref[...] / N - mean * mean
    inv = lax.rsqrt(var + 1e-5)
    h = (h1[...] - mean) * (inv * g[...]) + bt[...]
    h = jnp.where(h >= 0, h, 0.01 * h)
    y_ref[...] = jnp.dot(h, w2[...], preferred_element_type=_F32)
    z_ref[...] = jnp.dot(h, wl2[...], preferred_element_type=_F32) + b2l[...]


def _dense_b(h1, s1, q1, gamma, beta, W2, Wl2, bl2):
    return pl.pallas_call(
        _dense_b_body,
        grid=(_G,),
        in_specs=[
            pl.BlockSpec((_R, D_HID), lambda i: (i, 0)),
            pl.BlockSpec((1, D_HID), lambda i: (0, 0)),
            pl.BlockSpec((1, D_HID), lambda i: (0, 0)),
            pl.BlockSpec((1, D_HID), lambda i: (0, 0)),
            pl.BlockSpec((1, D_HID), lambda i: (0, 0)),
            pl.BlockSpec((D_HID, D_OUT), lambda i: (0, 0)),
            pl.BlockSpec((D_HID, D_OUT), lambda i: (0, 0)),
            pl.BlockSpec((1, D_OUT), lambda i: (0, 0)),
        ],
        out_specs=[
            pl.BlockSpec((_R, D_OUT), lambda i: (i, 0)),
            pl.BlockSpec((_R, D_OUT), lambda i: (i, 0)),
        ],
        out_shape=[
            jax.ShapeDtypeStruct((N, D_OUT), _F32),
            jax.ShapeDtypeStruct((N, D_OUT), _F32),
        ],
    )(h1, s1, q1, gamma, beta, W2, Wl2, bl2)


def _dense_c_body(a0, a1, y, z, d0, d1, b2, out_ref):
    scale = 1.0 / (d0[0] + d1[0] + 1.0)
    out_ref[...] = (a0[0] + a1[0] + y[...]) * scale + z[...] + b2[...]


def _dense_c(a0, a1, y, z, d0, d1, b2):
    return pl.pallas_call(
        _dense_c_body,
        grid=(_G,),
        in_specs=[
            pl.BlockSpec((1, _R, D_OUT), lambda i: (0, i, 0)),
            pl.BlockSpec((1, _R, D_OUT), lambda i: (1, i, 0)),
            pl.BlockSpec((_R, D_OUT), lambda i: (i, 0)),
            pl.BlockSpec((_R, D_OUT), lambda i: (i, 0)),
            pl.BlockSpec((1, _R, 1), lambda i: (0, i, 0)),
            pl.BlockSpec((1, _R, 1), lambda i: (1, i, 0)),
            pl.BlockSpec((1, D_OUT), lambda i: (0, 0)),
        ],
        out_specs=pl.BlockSpec((_R, D_OUT), lambda i: (i, 0)),
        out_shape=jax.ShapeDtypeStruct((N, D_OUT), _F32),
    )(a0, a1, y, z, d0, d1, b2)


def kernel(features, edge_index, W1, b1, Wl1, bl1, gamma, beta, W2, b2, Wl2,
           bl2):
    src = edge_index[0].reshape(NW, NG, GC, C)
    dst = edge_index[1].reshape(NW, NG, GC, C)
    rows = jnp.arange(NP, dtype=jnp.int32).reshape(NS, RPT // C, C)
    aggA, degw = _seg_sum_deg(features, rows, src, dst)
    deg3 = degw.reshape(NC, NP, 1)
    h1, s1, q1 = _dense_a(features, aggA, aggA, deg3, deg3, W1, Wl1,
                          (b1 + bl1).reshape(1, -1))
    y, z = _dense_b(h1, s1, q1, gamma.reshape(1, -1), beta.reshape(1, -1),
                    W2, Wl2, bl2.reshape(1, -1))
    (aggB,) = _seg_sum(y, rows, src, dst)
    out = _dense_c(aggB, aggB, y, z, deg3, deg3, b2.reshape(1, -1))
    return out
